# baseline (speedup 1.0000x reference)
import functools

import jax
import jax.numpy as jnp
from jax import lax
from jax.experimental import pallas as pl
from jax.experimental.pallas import tpu as pltpu
from jax.experimental.pallas import tpu_sc as plsc

N = 10000
NP = 10240
E = 320000
NC, NS = 2, 16
NW = NC * NS
CHUNK = 128
NCHUNKS = 80
EPW = NCHUNKS * CHUNK
EPAD = EPW * NW
NB = 4
ROWS_PT = NP // NS
HR = NP // 128
HR_PT = HR // NS

_MESH = dict(core_axis_name="c", subcore_axis_name="s", num_cores=NC,
             num_subcores=NS)

SLOW_CORE = 0
TOT_CHUNKS = EPAD // CHUNK


def _chunk_layout(c, s, cs, cf):
  if SLOW_CORE == 0:
    n = jnp.where(c == 0, cs, cf)
    base = jnp.where(c == 0, s * cs, NS * cs + s * cf)
  else:
    n = jnp.where(c == 0, cf, cs)
    base = jnp.where(c == 0, s * cf, NS * cf + s * cs)
  return n, base


def _deg_call(dst_pad, zeros_np, ones_chunk):
  mesh = plsc.VectorSubcoreMesh(**_MESH)
  SEG = NP // NS

  GRP = 8
  CS, CF = 56, 104

  @functools.partial(
      pl.kernel,
      out_type=jax.ShapeDtypeStruct((NC, NP), jnp.float32),
      mesh=mesh,
      scratch_types=[
          pltpu.VMEM((CF, CHUNK), jnp.int32),
          pltpu.VMEM((CHUNK,), jnp.float32),
          pltpu.VMEM_SHARED((NP,), jnp.float32),
          pltpu.SemaphoreType.DMA,
      ],
  )
  def deg_kernel(dst_hbm, z_hbm, ones_hbm, out_hbm, didx, ones_v, shist, sem):
    c = lax.axis_index("c")
    s = lax.axis_index("s")
    n, base = _chunk_layout(c, s, CS, CF)
    stage = jnp.minimum(base, TOT_CHUNKS - CF)
    off = base - stage
    pltpu.sync_copy(ones_hbm, ones_v)
    pltpu.sync_copy(dst_hbm.at[pl.ds(stage, CF)], didx)
    pltpu.sync_copy(z_hbm, shist.at[pl.ds(s * SEG, SEG)])
    plsc.subcore_barrier()

    def body(i, carry):
      for j in range(GRP):
        pltpu.async_copy(ones_v, shist.at[didx.at[off + i * GRP + j]], sem,
                         add=True)
      for j in range(GRP):
        pltpu.make_async_copy(ones_v, shist.at[didx.at[0]], sem).wait()
      return carry

    lax.fori_loop(0, n // GRP, body, 0)
    plsc.subcore_barrier()
    pltpu.sync_copy(shist.at[pl.ds(s * SEG, SEG)],
                    out_hbm.at[c, pl.ds(s * SEG, SEG)])

  return deg_kernel(dst_pad, zeros_np, ones_chunk)


def _aggregate_call(hp, src_pad, dst_pad, zeros_rows, d_model, R, CS, CF):
  NI = 2 * R
  mesh = plsc.VectorSubcoreMesh(**_MESH)

  @functools.partial(
      pl.kernel,
      out_type=jax.ShapeDtypeStruct((NC, NP, d_model), jnp.float32),
      mesh=mesh,
      scratch_types=[
          pltpu.VMEM((NI, CHUNK), jnp.int32),
          pltpu.VMEM((NI, CHUNK), jnp.int32),
          pltpu.VMEM((R, CHUNK, d_model), jnp.float32),
          pltpu.VMEM_SHARED((NP, d_model), jnp.float32),
          [pltpu.SemaphoreType.DMA] * NI,
          [pltpu.SemaphoreType.DMA] * R,
          [pltpu.SemaphoreType.DMA] * R,
      ],
      compiler_params=pltpu.CompilerParams(use_tc_tiling_on_sc=False),
  )
  def agg_kernel(hp_hbm, src_hbm, dst_hbm, z_hbm, out_hbm,
                 sidx, didx, rows, acc, isems, gsems, ssems):
    c = lax.axis_index("c")
    s = lax.axis_index("s")
    n, base = _chunk_layout(c, s, CS, CF)
    pltpu.sync_copy(z_hbm, acc.at[pl.ds(s * ROWS_PT, ROWS_PT)])
    plsc.subcore_barrier()

    def load_idx(q, g):
      pltpu.async_copy(src_hbm.at[base + g], sidx.at[q], isems[q])
      pltpu.async_copy(dst_hbm.at[base + g], didx.at[q], isems[q])

    def wait_idx(q):
      pltpu.make_async_copy(src_hbm.at[0], sidx.at[q], isems[q]).wait()
      pltpu.make_async_copy(src_hbm.at[0], didx.at[q], isems[q]).wait()

    def gather(q, rb):
      pltpu.async_copy(hp_hbm.at[sidx.at[q]], rows.at[rb], gsems[rb])

    for q in range(NI):
      load_idx(q, q)
    for b in range(R):
      wait_idx(b)
      gather(b, b)

    def body(i, carry):
      for b in range(NI):
        g = i * NI + b
        rb = b % R
        pltpu.make_async_copy(hp_hbm.at[sidx.at[0]], rows.at[rb],
                              gsems[rb]).wait()
        pltpu.async_copy(rows.at[rb], acc.at[didx.at[b]], ssems[rb], add=True)
        pltpu.make_async_copy(rows.at[rb], acc.at[didx.at[0]],
                              ssems[rb]).wait()
        @pl.when(g + NI < n)
        def _():
          load_idx(b, g + NI)

        @pl.when(g + R < n)
        def _():
          wait_idx((b + R) % NI)
          gather((b + R) % NI, rb)
      return carry

    lax.fori_loop(0, n // NI, body, 0)
    plsc.subcore_barrier()
    pltpu.sync_copy(acc.at[pl.ds(s * ROWS_PT, ROWS_PT)],
                    out_hbm.at[c, pl.ds(s * ROWS_PT, ROWS_PT)])

  return agg_kernel(hp, src_pad, dst_pad, zeros_rows)


_BR = 1280


def _row_spec(d):
  return pl.BlockSpec((_BR, d), lambda i: (i, 0))


def _full_spec(r, d):
  return pl.BlockSpec((r, d), lambda i: (0, 0))


def _tc1_call(deg_col, x_pad, W1):
  def body(deg_ref, x_ref, w_ref, hp_ref):
    dinv = lax.rsqrt(deg_ref[...] + 1.0)
    h = jnp.dot(x_ref[...], w_ref[...], preferred_element_type=jnp.float32)
    hp_ref[...] = dinv * h

  return pl.pallas_call(
      body,
      grid=(NP // _BR,),
      in_specs=[_row_spec(1), _row_spec(128), _full_spec(128, 128)],
      out_specs=_row_spec(128),
      out_shape=jax.ShapeDtypeStruct((NP, 128), jnp.float32),
  )(deg_col, x_pad, W1)


def _tc2_call(deg_col, p0, p1, hp1, W2, b1_row):
  def body(deg_ref, p0_ref, p1_ref, hp1_ref, w_ref, b_ref, hp2_ref):
    dinv = lax.rsqrt(deg_ref[...] + 1.0)
    a = dinv * (p0_ref[...] + p1_ref[...] + hp1_ref[...]) + b_ref[...]
    a = jnp.maximum(a, 0.0)
    h2 = jnp.dot(a, w_ref[...], preferred_element_type=jnp.float32)
    hp2_ref[...] = dinv * h2

  return pl.pallas_call(
      body,
      grid=(NP // _BR,),
      in_specs=[_row_spec(1), _row_spec(128), _row_spec(128), _row_spec(128),
                _full_spec(128, 64), _full_spec(1, 128)],
      out_specs=_row_spec(64),
      out_shape=jax.ShapeDtypeStruct((NP, 64), jnp.float32),
  )(deg_col, p0, p1, hp1, W2, b1_row)


def _tc3_call(deg_col, p0, p1, hp2, b2_row):
  def body(deg_ref, p0_ref, p1_ref, hp2_ref, b_ref, out_ref):
    dinv = lax.rsqrt(deg_ref[...] + 1.0)
    z = dinv * (p0_ref[...] + p1_ref[...] + hp2_ref[...]) + b_ref[...]
    m = jnp.max(z, axis=1, keepdims=True)
    e = jnp.exp(z - m)
    lse = jnp.log(jnp.sum(e, axis=1, keepdims=True)) + m
    out_ref[...] = z - lse

  return pl.pallas_call(
      body,
      grid=(NP // _BR,),
      in_specs=[_row_spec(1), _row_spec(64), _row_spec(64), _row_spec(64),
                _full_spec(1, 64)],
      out_specs=_row_spec(64),
      out_shape=jax.ShapeDtypeStruct((NP, 64), jnp.float32),
  )(deg_col, p0, p1, hp2, b2_row)


def kernel(x, edge_index, W1, b1, W2, b2):
  src = edge_index[0].astype(jnp.int32)
  dst = edge_index[1].astype(jnp.int32)
  pad = EPAD - E
  src_pad = jnp.concatenate([src, jnp.full((pad,), N, jnp.int32)])
  src_pad = src_pad.reshape(EPAD // CHUNK, CHUNK)
  dst_pad = jnp.concatenate([dst, jnp.full((pad,), N, jnp.int32)])
  dst_pad = dst_pad.reshape(EPAD // CHUNK, CHUNK)
  x_pad = jnp.pad(x, ((0, NP - N), (0, 0)))

  zeros_np = jnp.zeros((NP // NS,), jnp.float32)
  ones_chunk = jnp.ones((CHUNK,), jnp.float32)
  deg_parts = _deg_call(dst_pad, zeros_np, ones_chunk)
  deg_col = (deg_parts[0] + deg_parts[1]).reshape(NP, 1)

  hp1 = _tc1_call(deg_col, x_pad, W1)
  z128 = jnp.zeros((ROWS_PT, 128), jnp.float32)
  parts1 = _aggregate_call(hp1, src_pad, dst_pad, z128, 128, R=2, CS=36,
                           CF=124)
  hp2 = _tc2_call(deg_col, parts1[0], parts1[1], hp1, W2, b1.reshape(1, 128))
  z64 = jnp.zeros((ROWS_PT, 64), jnp.float32)
  parts2 = _aggregate_call(hp2, src_pad, dst_pad, z64, 64, R=4, CS=24, CF=136)
  out = _tc3_call(deg_col, parts2[0], parts2[1], hp2, b2.reshape(1, 64))
  return out[:N]


# speedup vs baseline: 1.1296x; 1.1296x over previous
import functools

import jax
import jax.numpy as jnp
from jax import lax
from jax.experimental import pallas as pl
from jax.experimental.pallas import tpu as pltpu
from jax.experimental.pallas import tpu_sc as plsc

N = 10000
NP = 10240
E = 320000
NC, NS = 2, 16
NW = NC * NS
CHUNK = 128
NCHUNKS = 80
EPW = NCHUNKS * CHUNK
EPAD = EPW * NW
NB = 4
ROWS_PT = NP // NS
HR = NP // 128
HR_PT = HR // NS

_MESH = dict(core_axis_name="c", subcore_axis_name="s", num_cores=NC,
             num_subcores=NS)

SLOW_CORE = 1
TOT_CHUNKS = EPAD // CHUNK


def _chunk_layout(c, s, cs, cf):
  if SLOW_CORE == 0:
    n = jnp.where(c == 0, cs, cf)
    base = jnp.where(c == 0, s * cs, NS * cs + s * cf)
  else:
    n = jnp.where(c == 0, cf, cs)
    base = jnp.where(c == 0, s * cf, NS * cf + s * cs)
  return n, base


def _deg_call(dst_pad, zeros_np, ones_chunk):
  mesh = plsc.VectorSubcoreMesh(**_MESH)
  SEG = NP // NS

  GRP = 8
  CS, CF = 56, 104

  @functools.partial(
      pl.kernel,
      out_type=jax.ShapeDtypeStruct((NC, NP), jnp.float32),
      mesh=mesh,
      scratch_types=[
          pltpu.VMEM((CF, CHUNK), jnp.int32),
          pltpu.VMEM((CHUNK,), jnp.float32),
          pltpu.VMEM_SHARED((NP,), jnp.float32),
          pltpu.SemaphoreType.DMA,
      ],
  )
  def deg_kernel(dst_hbm, z_hbm, ones_hbm, out_hbm, didx, ones_v, shist, sem):
    c = lax.axis_index("c")
    s = lax.axis_index("s")
    n, base = _chunk_layout(c, s, CS, CF)
    stage = jnp.minimum(base, TOT_CHUNKS - CF)
    off = base - stage
    pltpu.sync_copy(ones_hbm, ones_v)
    pltpu.sync_copy(dst_hbm.at[pl.ds(stage, CF)], didx)
    pltpu.sync_copy(z_hbm, shist.at[pl.ds(s * SEG, SEG)])
    plsc.subcore_barrier()

    def body(i, carry):
      for j in range(GRP):
        pltpu.async_copy(ones_v, shist.at[didx.at[off + i * GRP + j]], sem,
                         add=True)
      for j in range(GRP):
        pltpu.make_async_copy(ones_v, shist.at[didx.at[0]], sem).wait()
      return carry

    lax.fori_loop(0, n // GRP, body, 0)
    plsc.subcore_barrier()
    pltpu.sync_copy(shist.at[pl.ds(s * SEG, SEG)],
                    out_hbm.at[c, pl.ds(s * SEG, SEG)])

  return deg_kernel(dst_pad, zeros_np, ones_chunk)


def _aggregate_call(hp, src_pad, dst_pad, zeros_rows, d_model, R, CS, CF):
  NI = 2 * R
  mesh = plsc.VectorSubcoreMesh(**_MESH)

  @functools.partial(
      pl.kernel,
      out_type=jax.ShapeDtypeStruct((NC, NP, d_model), jnp.float32),
      mesh=mesh,
      scratch_types=[
          pltpu.VMEM((NI, CHUNK), jnp.int32),
          pltpu.VMEM((NI, CHUNK), jnp.int32),
          pltpu.VMEM((R, CHUNK, d_model), jnp.float32),
          pltpu.VMEM_SHARED((NP, d_model), jnp.float32),
          [pltpu.SemaphoreType.DMA] * NI,
          [pltpu.SemaphoreType.DMA] * R,
          [pltpu.SemaphoreType.DMA] * R,
      ],
      compiler_params=pltpu.CompilerParams(use_tc_tiling_on_sc=False),
  )
  def agg_kernel(hp_hbm, src_hbm, dst_hbm, z_hbm, out_hbm,
                 sidx, didx, rows, acc, isems, gsems, ssems):
    c = lax.axis_index("c")
    s = lax.axis_index("s")
    n, base = _chunk_layout(c, s, CS, CF)
    pltpu.sync_copy(z_hbm, acc.at[pl.ds(s * ROWS_PT, ROWS_PT)])
    plsc.subcore_barrier()

    def load_idx(q, g):
      pltpu.async_copy(src_hbm.at[base + g], sidx.at[q], isems[q])
      pltpu.async_copy(dst_hbm.at[base + g], didx.at[q], isems[q])

    def wait_idx(q):
      pltpu.make_async_copy(src_hbm.at[0], sidx.at[q], isems[q]).wait()
      pltpu.make_async_copy(src_hbm.at[0], didx.at[q], isems[q]).wait()

    def gather(q, rb):
      pltpu.async_copy(hp_hbm.at[sidx.at[q]], rows.at[rb], gsems[rb])

    for q in range(NI):
      load_idx(q, q)
    for b in range(R):
      wait_idx(b)
      gather(b, b)

    def body(i, carry):
      for b in range(NI):
        g = i * NI + b
        rb = b % R
        pltpu.make_async_copy(hp_hbm.at[sidx.at[0]], rows.at[rb],
                              gsems[rb]).wait()
        pltpu.async_copy(rows.at[rb], acc.at[didx.at[b]], ssems[rb], add=True)
        pltpu.make_async_copy(rows.at[rb], acc.at[didx.at[0]],
                              ssems[rb]).wait()
        @pl.when(g + NI < n)
        def _():
          load_idx(b, g + NI)

        @pl.when(g + R < n)
        def _():
          wait_idx((b + R) % NI)
          gather((b + R) % NI, rb)
      return carry

    lax.fori_loop(0, n // NI, body, 0)
    plsc.subcore_barrier()
    pltpu.sync_copy(acc.at[pl.ds(s * ROWS_PT, ROWS_PT)],
                    out_hbm.at[c, pl.ds(s * ROWS_PT, ROWS_PT)])

  return agg_kernel(hp, src_pad, dst_pad, zeros_rows)


_BR = 1280


def _row_spec(d):
  return pl.BlockSpec((_BR, d), lambda i: (i, 0))


def _full_spec(r, d):
  return pl.BlockSpec((r, d), lambda i: (0, 0))


def _tc1_call(deg_col, x_pad, W1):
  def body(deg_ref, x_ref, w_ref, hp_ref):
    dinv = lax.rsqrt(deg_ref[...] + 1.0)
    h = jnp.dot(x_ref[...], w_ref[...], preferred_element_type=jnp.float32)
    hp_ref[...] = dinv * h

  return pl.pallas_call(
      body,
      grid=(NP // _BR,),
      in_specs=[_row_spec(1), _row_spec(128), _full_spec(128, 128)],
      out_specs=_row_spec(128),
      out_shape=jax.ShapeDtypeStruct((NP, 128), jnp.float32),
  )(deg_col, x_pad, W1)


def _tc2_call(deg_col, p0, p1, hp1, W2, b1_row):
  def body(deg_ref, p0_ref, p1_ref, hp1_ref, w_ref, b_ref, hp2_ref):
    dinv = lax.rsqrt(deg_ref[...] + 1.0)
    a = dinv * (p0_ref[...] + p1_ref[...] + hp1_ref[...]) + b_ref[...]
    a = jnp.maximum(a, 0.0)
    h2 = jnp.dot(a, w_ref[...], preferred_element_type=jnp.float32)
    hp2_ref[...] = dinv * h2

  return pl.pallas_call(
      body,
      grid=(NP // _BR,),
      in_specs=[_row_spec(1), _row_spec(128), _row_spec(128), _row_spec(128),
                _full_spec(128, 64), _full_spec(1, 128)],
      out_specs=_row_spec(64),
      out_shape=jax.ShapeDtypeStruct((NP, 64), jnp.float32),
  )(deg_col, p0, p1, hp1, W2, b1_row)


def _tc3_call(deg_col, p0, p1, hp2, b2_row):
  def body(deg_ref, p0_ref, p1_ref, hp2_ref, b_ref, out_ref):
    dinv = lax.rsqrt(deg_ref[...] + 1.0)
    z = dinv * (p0_ref[...] + p1_ref[...] + hp2_ref[...]) + b_ref[...]
    m = jnp.max(z, axis=1, keepdims=True)
    e = jnp.exp(z - m)
    lse = jnp.log(jnp.sum(e, axis=1, keepdims=True)) + m
    out_ref[...] = z - lse

  return pl.pallas_call(
      body,
      grid=(NP // _BR,),
      in_specs=[_row_spec(1), _row_spec(64), _row_spec(64), _row_spec(64),
                _full_spec(1, 64)],
      out_specs=_row_spec(64),
      out_shape=jax.ShapeDtypeStruct((NP, 64), jnp.float32),
  )(deg_col, p0, p1, hp2, b2_row)


def kernel(x, edge_index, W1, b1, W2, b2):
  src = edge_index[0].astype(jnp.int32)
  dst = edge_index[1].astype(jnp.int32)
  pad = EPAD - E
  src_pad = jnp.concatenate([src, jnp.full((pad,), N, jnp.int32)])
  src_pad = src_pad.reshape(EPAD // CHUNK, CHUNK)
  dst_pad = jnp.concatenate([dst, jnp.full((pad,), N, jnp.int32)])
  dst_pad = dst_pad.reshape(EPAD // CHUNK, CHUNK)
  x_pad = jnp.pad(x, ((0, NP - N), (0, 0)))

  zeros_np = jnp.zeros((NP // NS,), jnp.float32)
  ones_chunk = jnp.ones((CHUNK,), jnp.float32)
  deg_parts = _deg_call(dst_pad, zeros_np, ones_chunk)
  deg_col = (deg_parts[0] + deg_parts[1]).reshape(NP, 1)

  hp1 = _tc1_call(deg_col, x_pad, W1)
  z128 = jnp.zeros((ROWS_PT, 128), jnp.float32)
  parts1 = _aggregate_call(hp1, src_pad, dst_pad, z128, 128, R=2, CS=36,
                           CF=124)
  hp2 = _tc2_call(deg_col, parts1[0], parts1[1], hp1, W2, b1.reshape(1, 128))
  z64 = jnp.zeros((ROWS_PT, 64), jnp.float32)
  parts2 = _aggregate_call(hp2, src_pad, dst_pad, z64, 64, R=4, CS=24, CF=136)
  out = _tc3_call(deg_col, parts2[0], parts2[1], hp2, b2.reshape(1, 64))
  return out[:N]
